# Initial kernel scaffold; baseline (speedup 1.0000x reference)
import jax
import jax.numpy as jnp
from jax.experimental import pallas as pl


def kernel(node_feats, edge_index, edge_attrs, edge_feats, W_up, W1, W2, W3, W4, W_lin):
    raise NotImplementedError("write your pallas kernel here")


# speedup vs baseline: 1.2505x; 1.2505x over previous
import functools

import jax
import jax.numpy as jnp
from jax import lax
from jax.experimental import pallas as pl
from jax.experimental.pallas import tpu as pltpu
from jax.experimental.pallas import tpu_sc as plsc

N = 10000
E = 320000
D = 128
D_EDGE = 16
HIDDEN = 64
AVG_NEIGH = 32.0

NW = 32
CH = 128
CG = 8
NG = 10
CJ = CG * NG
E_PAD = NW * CJ * CH
PACK = 4
BLK_PK = 128
N_BLKS = E_PAD // (PACK * BLK_PK)
REAL_BLKS = E // (PACK * BLK_PK)
NSUB = 16
N_PAD = 10240
NPS = N_PAD // NSUB


def _matmul_body(a_ref, w_ref, o_ref):
    o_ref[...] = jnp.dot(a_ref[...], w_ref[...],
                         preferred_element_type=jnp.float32)


def _final_body(p_ref, w_ref, o_ref):
    o_ref[...] = jnp.dot(p_ref[0] + p_ref[1], w_ref[...],
                         preferred_element_type=jnp.float32)


def _mlp_body(ef_ref, a_ref, w1_ref, w2_ref, w3_ref, w4_ref, e4_ref, o_ref):
    i = pl.program_id(0)
    ef = ef_ref[...]
    h = jax.nn.silu(jnp.dot(ef, w1_ref[...],
                            preferred_element_type=jnp.float32))
    h = jax.nn.silu(jnp.dot(h, w2_ref[...],
                            preferred_element_type=jnp.float32))
    h = jax.nn.silu(jnp.dot(h, w3_ref[...],
                            preferred_element_type=jnp.float32))
    ab = jnp.dot(a_ref[...], e4_ref[...], preferred_element_type=jnp.float32)
    w = jnp.dot(h * ab, w4_ref[...], preferred_element_type=jnp.float32)
    o_ref[...] = w * jnp.where(i < REAL_BLKS, 1.0, 0.0)


def _sc_body(x_hbm, wm_hbm, idx_hbm, zeros_hbm, out_hbm,
             idx_v, rows_v, wm_v, acc, sem_g, sem_w):
    cid = lax.axis_index("c")
    sid = lax.axis_index("s")
    tile = cid * NSUB + sid
    nslice = pl.ds(sid * NPS, NPS)
    pltpu.sync_copy(zeros_hbm.at[nslice], acc.at[nslice])
    plsc.subcore_barrier()
    base = tile * (CJ * CH)

    @pl.loop(0, NG)
    def _group(g):
        pltpu.sync_copy(idx_hbm.at[tile, g], idx_v)

        @pl.loop(0, CG)
        def _chunk(k):
            cw = pltpu.async_copy(
                wm_hbm.at[pl.ds(base + (g * CG + k) * CH, CH)], wm_v, sem_w)
            cg = pltpu.async_copy(x_hbm.at[idx_v.at[0, k]], rows_v, sem_g)
            cw.wait()
            cg.wait()

            @pl.loop(0, CH)
            def _row(r):
                for c in range(8):
                    sl = pl.ds(c * 16, 16)
                    rows_v[r, sl] = rows_v[r, sl] * wm_v[r, sl]

            pltpu.sync_copy(rows_v, acc.at[idx_v.at[1, k]], add=True)

    plsc.subcore_barrier()
    pltpu.sync_copy(acc.at[nslice], out_hbm.at[cid, nslice])


def kernel(node_feats, edge_index, edge_attrs, edge_feats,
           W_up, W1, W2, W3, W4, W_lin):
    f32 = jnp.float32
    W_up_s = W_up * (1.0 / jnp.sqrt(f32(D)))
    W_lin_s = W_lin * (1.0 / (jnp.sqrt(f32(D)) * AVG_NEIGH))
    W1_s = W1 * (1.0 / jnp.sqrt(f32(D_EDGE)))
    W2_s = W2 * (1.0 / jnp.sqrt(f32(HIDDEN)))
    W3_s = W3 * (1.0 / jnp.sqrt(f32(HIDDEN)))
    W4_s = W4 * (1.0 / jnp.sqrt(f32(HIDDEN)))

    def bdiag(w, reps):
        rows, cols = w.shape
        out = jnp.zeros((rows * reps, cols * reps), f32)
        for k in range(reps):
            out = out.at[k * rows:(k + 1) * rows,
                         k * cols:(k + 1) * cols].set(w)
        return out

    W1b = bdiag(W1_s, PACK)
    W2b = bdiag(W2_s, PACK)
    W3b = bdiag(W3_s, PACK)
    W4b = bdiag(W4_s, PACK)
    E4 = bdiag(jnp.ones((1, HIDDEN), f32), PACK)

    x = pl.pallas_call(
        _matmul_body,
        out_shape=jax.ShapeDtypeStruct((N, D), f32),
    )(node_feats, W_up_s)

    ef_pk = edge_feats.reshape(E // PACK, D_EDGE * PACK)
    a_pk = edge_attrs.reshape(E // PACK, PACK)
    full = lambda a: pl.BlockSpec(a.shape, lambda i: (0, 0))
    wm_pk = pl.pallas_call(
        _mlp_body,
        grid=(N_BLKS,),
        in_specs=[
            pl.BlockSpec((BLK_PK, D_EDGE * PACK),
                         lambda i: (jnp.minimum(i, REAL_BLKS - 1), 0)),
            pl.BlockSpec((BLK_PK, PACK),
                         lambda i: (jnp.minimum(i, REAL_BLKS - 1), 0)),
            full(W1b), full(W2b), full(W3b), full(W4b), full(E4),
        ],
        out_specs=pl.BlockSpec((BLK_PK, D * PACK), lambda i: (i, 0)),
        out_shape=jax.ShapeDtypeStruct((E_PAD // PACK, D * PACK), f32),
    )(ef_pk, a_pk, W1b, W2b, W3b, W4b, E4)
    wm = wm_pk.reshape(E_PAD, D)

    pad = E_PAD - E
    send = jnp.concatenate(
        [edge_index[0], jnp.zeros((pad,), jnp.int32)]).reshape(NW, NG, CG, CH)
    recv = jnp.concatenate(
        [edge_index[1], jnp.zeros((pad,), jnp.int32)]).reshape(NW, NG, CG, CH)
    idx = jnp.stack([send, recv], axis=2)
    zeros = jnp.zeros((N_PAD, D), f32)

    mesh = plsc.VectorSubcoreMesh(core_axis_name="c", subcore_axis_name="s")
    sc_call = functools.partial(
        pl.kernel,
        mesh=mesh,
        out_type=jax.ShapeDtypeStruct((2, N_PAD, D), f32),
        scratch_types=[
            pltpu.VMEM((2, CG, CH), jnp.int32),
            pltpu.VMEM((CH, D), f32),
            pltpu.VMEM((CH, D), f32),
            pltpu.VMEM_SHARED((N_PAD, D), f32),
            pltpu.SemaphoreType.DMA,
            pltpu.SemaphoreType.DMA,
        ],
    )(_sc_body)
    partials = sc_call(x, wm, idx, zeros)

    out_pad = pl.pallas_call(
        _final_body,
        out_shape=jax.ShapeDtypeStruct((N_PAD, D), f32),
    )(partials, W_lin_s)
    return out_pad[:N]


# speedup vs baseline: 1.3054x; 1.0439x over previous
import functools

import jax
import jax.numpy as jnp
from jax import lax
from jax.experimental import pallas as pl
from jax.experimental.pallas import tpu as pltpu
from jax.experimental.pallas import tpu_sc as plsc

N = 10000
E = 320000
D = 128
D_EDGE = 16
HIDDEN = 64
AVG_NEIGH = 32.0

NW = 32
CH = 128
CG = 8
NG = 10
CJ = CG * NG
E_PAD = NW * CJ * CH
CHP = CH // 4
PACK = 4
BLK_PK = 128
N_BLKS = E_PAD // (PACK * BLK_PK)
REAL_BLKS = E // (PACK * BLK_PK)
NSUB = 16
N_PAD = 10240
NPS = N_PAD // NSUB


def _matmul_body(a_ref, w_ref, o_ref):
    o_ref[...] = jnp.dot(a_ref[...], w_ref[...],
                         preferred_element_type=jnp.float32)


def _final_body(p_ref, w_ref, o_ref):
    o_ref[...] = jnp.dot(p_ref[0] + p_ref[1], w_ref[...],
                         preferred_element_type=jnp.float32)


def _mlp_body(ef_ref, a_ref, w1_ref, w2_ref, w3_ref, w4_ref, e4_ref, o_ref):
    i = pl.program_id(0)
    ef0 = ef_ref[...]
    ef = jnp.concatenate(
        [ef0[k * BLK_PK:(k + 1) * BLK_PK, :] for k in range(PACK)], axis=1)
    h = jax.nn.silu(jnp.dot(ef, w1_ref[...],
                            preferred_element_type=jnp.float32))
    h = jax.nn.silu(jnp.dot(h, w2_ref[...],
                            preferred_element_type=jnp.float32))
    h = jax.nn.silu(jnp.dot(h, w3_ref[...],
                            preferred_element_type=jnp.float32))
    a0 = a_ref[...]
    a4 = jnp.concatenate(
        [a0[k * BLK_PK:(k + 1) * BLK_PK, :] for k in range(PACK)], axis=1)
    ab = jnp.dot(a4, e4_ref[...], preferred_element_type=jnp.float32)
    w = jnp.dot(h * ab, w4_ref[...], preferred_element_type=jnp.float32)
    o_ref[...] = w * jnp.where(i < REAL_BLKS, 1.0, 0.0)


def _sc_body(x_hbm, wm_hbm, send_hbm, recv_hbm, zeros_hbm, out_hbm,
             idx_v, rows_v, wm_v, acc, sem_g, sem_w):
    cid = lax.axis_index("c")
    sid = lax.axis_index("s")
    tile = cid * NSUB + sid
    nslice = pl.ds(sid * NPS, NPS)
    pltpu.sync_copy(zeros_hbm.at[nslice], acc.at[nslice])
    plsc.subcore_barrier()
    pbase = tile * (CJ * CHP)

    @pl.loop(0, NG)
    def _group(g):
        pltpu.sync_copy(send_hbm.at[tile, g], idx_v.at[0])
        pltpu.sync_copy(recv_hbm.at[tile, g], idx_v.at[1])

        @pl.loop(0, CG)
        def _chunk(k):
            cw = pltpu.async_copy(
                wm_hbm.at[pl.ds(pbase + (g * CG + k) * CHP, CHP)],
                wm_v, sem_w)
            cg = pltpu.async_copy(x_hbm.at[idx_v.at[0, k]], rows_v, sem_g)
            cw.wait()
            cg.wait()

            @pl.loop(0, CHP)
            def _row(r):
                for kk in range(PACK):
                    for c in range(8):
                        sl = pl.ds(c * 16, 16)
                        slw = pl.ds(kk * D + c * 16, 16)
                        rows_v[r * PACK + kk, sl] = (
                            rows_v[r * PACK + kk, sl] * wm_v[r, slw])

            pltpu.sync_copy(rows_v, acc.at[idx_v.at[1, k]], add=True)

    plsc.subcore_barrier()
    pltpu.sync_copy(acc.at[nslice], out_hbm.at[cid, nslice])


def kernel(node_feats, edge_index, edge_attrs, edge_feats,
           W_up, W1, W2, W3, W4, W_lin):
    f32 = jnp.float32
    W_up_s = W_up * (1.0 / jnp.sqrt(f32(D)))
    W_lin_s = W_lin * (1.0 / (jnp.sqrt(f32(D)) * AVG_NEIGH))
    W1_s = W1 * (1.0 / jnp.sqrt(f32(D_EDGE)))
    W2_s = W2 * (1.0 / jnp.sqrt(f32(HIDDEN)))
    W3_s = W3 * (1.0 / jnp.sqrt(f32(HIDDEN)))
    W4_s = W4 * (1.0 / jnp.sqrt(f32(HIDDEN)))

    def bdiag(w, reps):
        rows, cols = w.shape
        out = jnp.zeros((rows * reps, cols * reps), f32)
        for k in range(reps):
            out = out.at[k * rows:(k + 1) * rows,
                         k * cols:(k + 1) * cols].set(w)
        return out

    W1b = bdiag(W1_s, PACK)
    W2b = bdiag(W2_s, PACK)
    W3b = bdiag(W3_s, PACK)
    W4b = bdiag(W4_s, PACK)
    E4 = bdiag(jnp.ones((1, HIDDEN), f32), PACK)

    x = pl.pallas_call(
        _matmul_body,
        out_shape=jax.ShapeDtypeStruct((N, D), f32),
    )(node_feats, W_up_s)

    full = lambda a: pl.BlockSpec(a.shape, lambda i: (0, 0))
    wm_pk = pl.pallas_call(
        _mlp_body,
        grid=(N_BLKS,),
        in_specs=[
            pl.BlockSpec((BLK_PK * PACK, D_EDGE),
                         lambda i: (jnp.minimum(i, REAL_BLKS - 1), 0)),
            pl.BlockSpec((BLK_PK * PACK, 1),
                         lambda i: (jnp.minimum(i, REAL_BLKS - 1), 0)),
            full(W1b), full(W2b), full(W3b), full(W4b), full(E4),
        ],
        out_specs=pl.BlockSpec((BLK_PK, D * PACK), lambda i: (i, 0)),
        out_shape=jax.ShapeDtypeStruct((E_PAD // PACK, D * PACK), f32),
    )(edge_feats, edge_attrs, W1b, W2b, W3b, W4b, E4)

    pad = E_PAD - E
    spread = jnp.arange(pad, dtype=jnp.int32) % N

    def to_packed(v):
        v = jnp.concatenate([v, spread])
        v = v.reshape(N_BLKS, PACK, BLK_PK // CHP, CHP).transpose(0, 2, 3, 1)
        return v.reshape(NW, NG, CG, CH)

    send = to_packed(edge_index[0])
    recv = to_packed(edge_index[1])
    zeros = jnp.zeros((N_PAD, D), f32)

    mesh = plsc.VectorSubcoreMesh(core_axis_name="c", subcore_axis_name="s")
    sc_call = functools.partial(
        pl.kernel,
        mesh=mesh,
        out_type=jax.ShapeDtypeStruct((2, N_PAD, D), f32),
        scratch_types=[
            pltpu.VMEM((2, CG, CH), jnp.int32),
            pltpu.VMEM((CH, D), f32),
            pltpu.VMEM((CHP, D * PACK), f32),
            pltpu.VMEM_SHARED((N_PAD, D), f32),
            pltpu.SemaphoreType.DMA,
            pltpu.SemaphoreType.DMA,
        ],
    )(_sc_body)
    partials = sc_call(x, wm_pk, send, recv, zeros)

    out_pad = pl.pallas_call(
        _final_body,
        out_shape=jax.ShapeDtypeStruct((N_PAD, D), f32),
    )(partials, W_lin_s)
    return out_pad[:N]


# speedup vs baseline: 1.4985x; 1.1479x over previous
import functools

import jax
import jax.numpy as jnp
from jax import lax
from jax.experimental import pallas as pl
from jax.experimental.pallas import tpu as pltpu
from jax.experimental.pallas import tpu_sc as plsc

N = 10000
E = 320000
D = 128
D_EDGE = 16
HIDDEN = 64
AVG_NEIGH = 32.0

NW = 32
CH = 128
CG = 8
NG = 10
CJ = CG * NG
E_PAD = NW * CJ * CH
CHP = CH // 4
PACK = 4
BLK_PK = 640
W_WIDE = BLK_PK // 2
N_BLKS = E_PAD // (PACK * BLK_PK)
REAL_BLKS = E // (PACK * BLK_PK)
NSUB = 16
N_PAD = 10240
NPS = N_PAD // NSUB


def _matmul_body(a_ref, w_ref, o_ref):
    o_ref[...] = jnp.dot(a_ref[...], w_ref[...],
                         preferred_element_type=jnp.float32)


def _final_body(p_ref, w_ref, o_ref):
    o_ref[...] = jnp.dot(p_ref[0] + p_ref[1], w_ref[...],
                         preferred_element_type=jnp.float32)


def _mlp_body(ef_ref, a_ref, w1_ref, w2_ref, w3_ref, w4_ref, e4_ref, o_ref):
    i = pl.program_id(0)
    ef0 = ef_ref[...]
    ef = jnp.concatenate([ef0[:, :64], ef0[:, 64:]], axis=0)
    h = jax.nn.silu(jnp.dot(ef, w1_ref[...],
                            preferred_element_type=jnp.float32))
    h = jax.nn.silu(jnp.dot(h, w2_ref[...],
                            preferred_element_type=jnp.float32))
    h = jax.nn.silu(jnp.dot(h, w3_ref[...],
                            preferred_element_type=jnp.float32))
    ab = jnp.dot(a_ref[...], e4_ref[...], preferred_element_type=jnp.float32)
    w = jnp.dot(h * ab, w4_ref[...], preferred_element_type=jnp.float32)
    o_ref[...] = w * jnp.where(i < REAL_BLKS, 1.0, 0.0)


def _sc_body(x_hbm, wm_hbm, send_hbm, recv_hbm, zeros_hbm, out_hbm,
             idx_v, rows_v, wm_v, acc, sem_g, sem_w):
    cid = lax.axis_index("c")
    sid = lax.axis_index("s")
    tile = cid * NSUB + sid
    nslice = pl.ds(sid * NPS, NPS)
    pltpu.sync_copy(zeros_hbm.at[nslice], acc.at[nslice])
    plsc.subcore_barrier()
    pbase = tile * (CJ * CHP)

    @pl.loop(0, NG)
    def _group(g):
        pltpu.sync_copy(send_hbm.at[tile, g], idx_v.at[0])
        pltpu.sync_copy(recv_hbm.at[tile, g], idx_v.at[1])

        @pl.loop(0, CG)
        def _chunk(k):
            cw = pltpu.async_copy(
                wm_hbm.at[pl.ds(pbase + (g * CG + k) * CHP, CHP)],
                wm_v, sem_w)
            cg = pltpu.async_copy(x_hbm.at[idx_v.at[0, k]], rows_v, sem_g)
            cw.wait()
            cg.wait()

            @pl.loop(0, CHP)
            def _row(r):
                for kk in range(PACK):
                    for c in range(8):
                        sl = pl.ds(c * 16, 16)
                        slw = pl.ds(kk * D + c * 16, 16)
                        rows_v[r * PACK + kk, sl] = (
                            rows_v[r * PACK + kk, sl] * wm_v[r, slw])

            pltpu.sync_copy(rows_v, acc.at[idx_v.at[1, k]], add=True)

    plsc.subcore_barrier()
    pltpu.sync_copy(acc.at[nslice], out_hbm.at[cid, nslice])


def kernel(node_feats, edge_index, edge_attrs, edge_feats,
           W_up, W1, W2, W3, W4, W_lin):
    f32 = jnp.float32
    W_up_s = W_up * (1.0 / jnp.sqrt(f32(D)))
    W_lin_s = W_lin * (1.0 / (jnp.sqrt(f32(D)) * AVG_NEIGH))
    W1_s = W1 * (1.0 / jnp.sqrt(f32(D_EDGE)))
    W2_s = W2 * (1.0 / jnp.sqrt(f32(HIDDEN)))
    W3_s = W3 * (1.0 / jnp.sqrt(f32(HIDDEN)))
    W4_s = W4 * (1.0 / jnp.sqrt(f32(HIDDEN)))

    def bdiag(w, reps):
        rows, cols = w.shape
        out = jnp.zeros((rows * reps, cols * reps), f32)
        for k in range(reps):
            out = out.at[k * rows:(k + 1) * rows,
                         k * cols:(k + 1) * cols].set(w)
        return out

    W1b = bdiag(W1_s, PACK)
    W2b = bdiag(W2_s, PACK)
    W3b = bdiag(W3_s, PACK)
    W4b = bdiag(W4_s, PACK)
    E4 = bdiag(jnp.ones((1, HIDDEN), f32), PACK)

    x = pl.pallas_call(
        _matmul_body,
        out_shape=jax.ShapeDtypeStruct((N, D), f32),
    )(node_feats, W_up_s)

    ef_wide = edge_feats.reshape(E // 8, 128)
    a_pk = (edge_attrs[:, 0]
            .reshape(REAL_BLKS, W_WIDE, 2, PACK)
            .transpose(0, 2, 1, 3)
            .reshape(REAL_BLKS * BLK_PK, PACK))
    full = lambda a: pl.BlockSpec(a.shape, lambda i: (0, 0))
    wm_pk = pl.pallas_call(
        _mlp_body,
        grid=(N_BLKS,),
        in_specs=[
            pl.BlockSpec((W_WIDE, 128),
                         lambda i: (jnp.minimum(i, REAL_BLKS - 1), 0)),
            pl.BlockSpec((BLK_PK, PACK),
                         lambda i: (jnp.minimum(i, REAL_BLKS - 1), 0)),
            full(W1b), full(W2b), full(W3b), full(W4b), full(E4),
        ],
        out_specs=pl.BlockSpec((BLK_PK, D * PACK), lambda i: (i, 0)),
        out_shape=jax.ShapeDtypeStruct((E_PAD // PACK, D * PACK), f32),
    )(ef_wide, a_pk, W1b, W2b, W3b, W4b, E4)

    pad = E_PAD - E
    spread = jnp.arange(pad, dtype=jnp.int32) % N

    def to_packed(v):
        v = jnp.concatenate([v, spread])
        v = v.reshape(N_BLKS, W_WIDE, 2, PACK).transpose(0, 2, 1, 3)
        v = v.reshape(N_BLKS, 2, W_WIDE // CHP, CHP, PACK)
        return v.reshape(NW, NG, CG, CH)

    send = to_packed(edge_index[0])
    recv = to_packed(edge_index[1])
    zeros = jnp.zeros((N_PAD, D), f32)

    mesh = plsc.VectorSubcoreMesh(core_axis_name="c", subcore_axis_name="s")
    sc_call = functools.partial(
        pl.kernel,
        mesh=mesh,
        out_type=jax.ShapeDtypeStruct((2, N_PAD, D), f32),
        scratch_types=[
            pltpu.VMEM((2, CG, CH), jnp.int32),
            pltpu.VMEM((CH, D), f32),
            pltpu.VMEM((CHP, D * PACK), f32),
            pltpu.VMEM_SHARED((N_PAD, D), f32),
            pltpu.SemaphoreType.DMA,
            pltpu.SemaphoreType.DMA,
        ],
    )(_sc_body)
    partials = sc_call(x, wm_pk, send, recv, zeros)

    out_pad = pl.pallas_call(
        _final_body,
        out_shape=jax.ShapeDtypeStruct((N_PAD, D), f32),
    )(partials, W_lin_s)
    return out_pad[:N]


# speedup vs baseline: 2.6620x; 1.7765x over previous
import functools

import jax
import jax.numpy as jnp
from jax import lax
from jax.experimental import pallas as pl
from jax.experimental.pallas import tpu as pltpu
from jax.experimental.pallas import tpu_sc as plsc

N = 10000
E = 320000
D = 128
D_EDGE = 16
HIDDEN = 64
AVG_NEIGH = 32.0

NW = 32
CH = 128
CG = 8
NG = 10
CJ = CG * NG
E_PAD = NW * CJ * CH
CHP = CH // 4
CPB = 20
PACK = 4
BLK_PK = 640
W_WIDE = BLK_PK // 2
N_BLKS = E_PAD // (PACK * BLK_PK)
REAL_BLKS = E // (PACK * BLK_PK)
NSUB = 16
N_PAD = 10240
NPS = N_PAD // NSUB


def _matmul_body(a_ref, w_ref, o_ref):
    o_ref[...] = jnp.dot(a_ref[...], w_ref[...],
                         preferred_element_type=jnp.float32)


def _final_body(p_ref, w_ref, o_ref):
    o_ref[...] = jnp.dot(p_ref[0] + p_ref[1], w_ref[...],
                         preferred_element_type=jnp.float32)


def _mlp_body(ef_ref, a_ref, w1_ref, w2_ref, w3_ref, w4_ref, e4_ref, o_ref):
    i = pl.program_id(0)
    ef0 = ef_ref[...]
    ef = jnp.concatenate(
        [ef0[k * BLK_PK:(k + 1) * BLK_PK, :] for k in range(PACK)], axis=1)
    h = jax.nn.silu(jnp.dot(ef, w1_ref[...],
                            preferred_element_type=jnp.float32))
    h = jax.nn.silu(jnp.dot(h, w2_ref[...],
                            preferred_element_type=jnp.float32))
    h = jax.nn.silu(jnp.dot(h, w3_ref[...],
                            preferred_element_type=jnp.float32))
    a0 = a_ref[...]
    a4 = jnp.concatenate(
        [a0[k * BLK_PK:(k + 1) * BLK_PK, :] for k in range(PACK)], axis=1)
    ab = jnp.dot(a4, e4_ref[...], preferred_element_type=jnp.float32)
    w = jnp.dot(h * ab, w4_ref[...], preferred_element_type=jnp.float32)
    o_ref[...] = w * jnp.where(i < REAL_BLKS, 1.0, 0.0)


def _sc_body(x_hbm, wm_hbm, send_hbm, recv_hbm, zeros_hbm, out_hbm,
             idx_v, rows_v, wm_v, acc, sem_g, sem_w):
    cid = lax.axis_index("c")
    sid = lax.axis_index("s")
    tile = cid * NSUB + sid
    nslice = pl.ds(sid * NPS, NPS)
    pltpu.sync_copy(zeros_hbm.at[nslice], acc.at[nslice])
    plsc.subcore_barrier()

    @pl.loop(0, NG)
    def _group(g):
        pltpu.sync_copy(send_hbm.at[tile, g], idx_v.at[0])
        pltpu.sync_copy(recv_hbm.at[tile, g], idx_v.at[1])

        @pl.loop(0, CG)
        def _chunk(kc):
            j = tile * CJ + g * CG + kc
            b = j // CPB
            rem = j - b * CPB
            kseg = rem // 5
            c5 = rem - kseg * 5
            cw = pltpu.async_copy(
                wm_hbm.at[pl.ds(b * BLK_PK + c5 * CH, CH),
                          pl.ds(kseg * D, D)],
                wm_v, sem_w)
            cg = pltpu.async_copy(x_hbm.at[idx_v.at[0, kc]], rows_v, sem_g)
            cw.wait()
            cg.wait()

            @pl.loop(0, CH)
            def _row(r):
                for c in range(8):
                    sl = pl.ds(c * 16, 16)
                    rows_v[r, sl] = rows_v[r, sl] * wm_v[r, sl]

            pltpu.sync_copy(rows_v, acc.at[idx_v.at[1, kc]], add=True)

    plsc.subcore_barrier()
    pltpu.sync_copy(acc.at[nslice], out_hbm.at[cid, nslice])


def kernel(node_feats, edge_index, edge_attrs, edge_feats,
           W_up, W1, W2, W3, W4, W_lin):
    f32 = jnp.float32
    W_up_s = W_up * (1.0 / jnp.sqrt(f32(D)))
    W_lin_s = W_lin * (1.0 / (jnp.sqrt(f32(D)) * AVG_NEIGH))
    W1_s = W1 * (1.0 / jnp.sqrt(f32(D_EDGE)))
    W2_s = W2 * (1.0 / jnp.sqrt(f32(HIDDEN)))
    W3_s = W3 * (1.0 / jnp.sqrt(f32(HIDDEN)))
    W4_s = W4 * (1.0 / jnp.sqrt(f32(HIDDEN)))

    def bdiag(w, reps):
        rows, cols = w.shape
        out = jnp.zeros((rows * reps, cols * reps), f32)
        for k in range(reps):
            out = out.at[k * rows:(k + 1) * rows,
                         k * cols:(k + 1) * cols].set(w)
        return out

    W1b = bdiag(W1_s, PACK)
    W2b = bdiag(W2_s, PACK)
    W3b = bdiag(W3_s, PACK)
    W4b = bdiag(W4_s, PACK)
    E4 = bdiag(jnp.ones((1, HIDDEN), f32), PACK)

    x = pl.pallas_call(
        _matmul_body,
        out_shape=jax.ShapeDtypeStruct((N, D), f32),
    )(node_feats, W_up_s)

    full = lambda a: pl.BlockSpec(a.shape, lambda i: (0, 0))
    wm_pk = pl.pallas_call(
        _mlp_body,
        grid=(N_BLKS,),
        in_specs=[
            pl.BlockSpec((PACK * BLK_PK, D_EDGE),
                         lambda i: (jnp.minimum(i, REAL_BLKS - 1), 0)),
            pl.BlockSpec((PACK * BLK_PK, 1),
                         lambda i: (jnp.minimum(i, REAL_BLKS - 1), 0)),
            full(W1b), full(W2b), full(W3b), full(W4b), full(E4),
        ],
        out_specs=pl.BlockSpec((BLK_PK, D * PACK), lambda i: (i, 0)),
        out_shape=jax.ShapeDtypeStruct((E_PAD // PACK, D * PACK), f32),
    )(edge_feats, edge_attrs, W1b, W2b, W3b, W4b, E4)

    pad = E_PAD - E
    spread = jnp.arange(pad, dtype=jnp.int32) % N

    send = jnp.concatenate([edge_index[0], spread]).reshape(NW, NG, CG, CH)
    recv = jnp.concatenate([edge_index[1], spread]).reshape(NW, NG, CG, CH)
    zeros = jnp.zeros((N_PAD, D), f32)

    mesh = plsc.VectorSubcoreMesh(core_axis_name="c", subcore_axis_name="s")
    sc_call = functools.partial(
        pl.kernel,
        mesh=mesh,
        out_type=jax.ShapeDtypeStruct((2, N_PAD, D), f32),
        scratch_types=[
            pltpu.VMEM((2, CG, CH), jnp.int32),
            pltpu.VMEM((CH, D), f32),
            pltpu.VMEM((CH, D), f32),
            pltpu.VMEM_SHARED((N_PAD, D), f32),
            pltpu.SemaphoreType.DMA,
            pltpu.SemaphoreType.DMA,
        ],
    )(_sc_body)
    partials = sc_call(x, wm_pk, send, recv, zeros)

    out_pad = pl.pallas_call(
        _final_body,
        out_shape=jax.ShapeDtypeStruct((N_PAD, D), f32),
    )(partials, W_lin_s)
    return out_pad[:N]


# speedup vs baseline: 3.1626x; 1.1880x over previous
import functools

import jax
import jax.numpy as jnp
from jax import lax
from jax.experimental import pallas as pl
from jax.experimental.pallas import tpu as pltpu
from jax.experimental.pallas import tpu_sc as plsc

N = 10000
E = 320000
D = 128
D_EDGE = 16
HIDDEN = 64
AVG_NEIGH = 32.0

NW = 32
CH = 128
CG = 8
NG = 10
CJ = CG * NG
E_PAD = NW * CJ * CH
CHP = CH // 4
CH2 = 64
CJ2 = 160
NP = 80
CPB2 = 40
PACK = 4
BLK_PK = 640
W_WIDE = BLK_PK // 2
N_BLKS = E_PAD // (PACK * BLK_PK)
REAL_BLKS = E // (PACK * BLK_PK)
NSUB = 16
N_PAD = 10240
NPS = N_PAD // NSUB


def _matmul_body(a_ref, w_ref, o_ref):
    o_ref[...] = jnp.dot(a_ref[...], w_ref[...],
                         preferred_element_type=jnp.float32)


def _final_body(p_ref, w_ref, o_ref):
    o_ref[...] = jnp.dot(p_ref[0] + p_ref[1], w_ref[...],
                         preferred_element_type=jnp.float32)


def _mlp_body(ef_ref, a_ref, w1_ref, w2_ref, w3_ref, w4_ref, e4_ref, o_ref):
    i = pl.program_id(0)
    ef0 = ef_ref[...]
    ef = jnp.concatenate(
        [ef0[k * BLK_PK:(k + 1) * BLK_PK, :] for k in range(PACK)], axis=1)
    h = jax.nn.silu(jnp.dot(ef, w1_ref[...],
                            preferred_element_type=jnp.float32))
    h = jax.nn.silu(jnp.dot(h, w2_ref[...],
                            preferred_element_type=jnp.float32))
    h = jax.nn.silu(jnp.dot(h, w3_ref[...],
                            preferred_element_type=jnp.float32))
    a0 = a_ref[...]
    a4 = jnp.concatenate(
        [a0[k * BLK_PK:(k + 1) * BLK_PK, :] for k in range(PACK)], axis=1)
    ab = jnp.dot(a4, e4_ref[...], preferred_element_type=jnp.float32)
    w = jnp.dot(h * ab, w4_ref[...], preferred_element_type=jnp.float32)
    o_ref[...] = w * jnp.where(i < REAL_BLKS, 1.0, 0.0)


def _sc_body(x_hbm, wm_hbm, send_hbm, recv_hbm, zeros_hbm, out_hbm,
             idx_v, rows_v, wm_v, acc, sem_i, sg0, sg1, sw0, sw1):
    cid = lax.axis_index("c")
    sid = lax.axis_index("s")
    tile = cid * NSUB + sid
    nslice = pl.ds(sid * NPS, NPS)
    pltpu.sync_copy(zeros_hbm.at[nslice], acc.at[nslice])
    plsc.subcore_barrier()
    sgs = (sg0, sg1)
    sws = (sw0, sw1)

    def issue(j, s):
        b = j // CPB2
        rem = j - b * CPB2
        kseg = rem // 10
        c10 = rem - kseg * 10
        cw = pltpu.async_copy(
            wm_hbm.at[pl.ds(b * BLK_PK + c10 * CH2, CH2),
                      pl.ds(kseg * D, D)],
            wm_v.at[s], sws[s])
        p = (j // 2) % 2
        cg = pltpu.async_copy(x_hbm.at[idx_v.at[p, 0, j % 2]],
                              rows_v.at[s], sgs[s])
        return cw, cg

    def process(j, s):
        pltpu.make_async_copy(wm_hbm.at[pl.ds(0, CH2), pl.ds(0, D)],
                              wm_v.at[s], sws[s]).wait()
        pltpu.make_async_copy(x_hbm.at[idx_v.at[0, 0, 0]],
                              rows_v.at[s], sgs[s]).wait()

        @pl.loop(0, CH2)
        def _row(r):
            for c in range(8):
                sl = pl.ds(c * 16, 16)
                rows_v[s, r, sl] = rows_v[s, r, sl] * wm_v[s, r, sl]

        p = (j // 2) % 2
        pltpu.sync_copy(rows_v.at[s], acc.at[idx_v.at[p, 1, j % 2]],
                        add=True)

    pltpu.sync_copy(send_hbm.at[tile, 0], idx_v.at[0, 0])
    pltpu.sync_copy(recv_hbm.at[tile, 0], idx_v.at[0, 1])
    issue(tile * CJ2 + 0, 0)
    issue(tile * CJ2 + 1, 1)

    @pl.loop(0, NP)
    def _pair(jj):
        p = jj % 2
        j0 = tile * CJ2 + jj * 2

        @pl.when(jj < NP - 1)
        def _pf():
            ci1 = pltpu.async_copy(send_hbm.at[tile, jj + 1],
                                   idx_v.at[1 - p, 0], sem_i)
            ci2 = pltpu.async_copy(recv_hbm.at[tile, jj + 1],
                                   idx_v.at[1 - p, 1], sem_i)

        process(j0, 0)

        @pl.when(jj < NP - 1)
        def _n0():
            pltpu.make_async_copy(send_hbm.at[tile, 0], idx_v.at[0, 0],
                                  sem_i).wait()
            pltpu.make_async_copy(recv_hbm.at[tile, 0], idx_v.at[0, 1],
                                  sem_i).wait()
            issue(j0 + 2, 0)

        process(j0 + 1, 1)

        @pl.when(jj < NP - 1)
        def _n1():
            issue(j0 + 3, 1)

    plsc.subcore_barrier()
    pltpu.sync_copy(acc.at[nslice], out_hbm.at[cid, nslice])


def kernel(node_feats, edge_index, edge_attrs, edge_feats,
           W_up, W1, W2, W3, W4, W_lin):
    f32 = jnp.float32
    W_up_s = W_up * (1.0 / jnp.sqrt(f32(D)))
    W_lin_s = W_lin * (1.0 / (jnp.sqrt(f32(D)) * AVG_NEIGH))
    W1_s = W1 * (1.0 / jnp.sqrt(f32(D_EDGE)))
    W2_s = W2 * (1.0 / jnp.sqrt(f32(HIDDEN)))
    W3_s = W3 * (1.0 / jnp.sqrt(f32(HIDDEN)))
    W4_s = W4 * (1.0 / jnp.sqrt(f32(HIDDEN)))

    def bdiag(w, reps):
        rows, cols = w.shape
        out = jnp.zeros((rows * reps, cols * reps), f32)
        for k in range(reps):
            out = out.at[k * rows:(k + 1) * rows,
                         k * cols:(k + 1) * cols].set(w)
        return out

    W1b = bdiag(W1_s, PACK)
    W2b = bdiag(W2_s, PACK)
    W3b = bdiag(W3_s, PACK)
    W4b = bdiag(W4_s, PACK)
    E4 = bdiag(jnp.ones((1, HIDDEN), f32), PACK)

    x = pl.pallas_call(
        _matmul_body,
        out_shape=jax.ShapeDtypeStruct((N, D), f32),
    )(node_feats, W_up_s)

    full = lambda a: pl.BlockSpec(a.shape, lambda i: (0, 0))
    wm_pk = pl.pallas_call(
        _mlp_body,
        grid=(N_BLKS,),
        in_specs=[
            pl.BlockSpec((PACK * BLK_PK, D_EDGE),
                         lambda i: (jnp.minimum(i, REAL_BLKS - 1), 0)),
            pl.BlockSpec((PACK * BLK_PK, 1),
                         lambda i: (jnp.minimum(i, REAL_BLKS - 1), 0)),
            full(W1b), full(W2b), full(W3b), full(W4b), full(E4),
        ],
        out_specs=pl.BlockSpec((BLK_PK, D * PACK), lambda i: (i, 0)),
        out_shape=jax.ShapeDtypeStruct((E_PAD // PACK, D * PACK), f32),
    )(edge_feats, edge_attrs, W1b, W2b, W3b, W4b, E4)

    pad = E_PAD - E
    spread = jnp.arange(pad, dtype=jnp.int32) % N

    send = jnp.concatenate([edge_index[0], spread]).reshape(NW, NP, 2, CH2)
    recv = jnp.concatenate([edge_index[1], spread]).reshape(NW, NP, 2, CH2)
    zeros = jnp.zeros((N_PAD, D), f32)

    mesh = plsc.VectorSubcoreMesh(core_axis_name="c", subcore_axis_name="s")
    sc_call = functools.partial(
        pl.kernel,
        mesh=mesh,
        out_type=jax.ShapeDtypeStruct((2, N_PAD, D), f32),
        scratch_types=[
            pltpu.VMEM((2, 2, 2, CH2), jnp.int32),
            pltpu.VMEM((2, CH2, D), f32),
            pltpu.VMEM((2, CH2, D), f32),
            pltpu.VMEM_SHARED((N_PAD, D), f32),
            pltpu.SemaphoreType.DMA,
            pltpu.SemaphoreType.DMA,
            pltpu.SemaphoreType.DMA,
            pltpu.SemaphoreType.DMA,
            pltpu.SemaphoreType.DMA,
        ],
    )(_sc_body)
    partials = sc_call(x, wm_pk, send, recv, zeros)

    out_pad = pl.pallas_call(
        _final_body,
        out_shape=jax.ShapeDtypeStruct((N_PAD, D), f32),
    )(partials, W_lin_s)
    return out_pad[:N]
